# baseline (speedup 1.0000x reference)
import functools

import jax
import jax.numpy as jnp
from jax import lax
from jax.experimental import pallas as pl
from jax.experimental.pallas import tpu as pltpu
from jax.experimental.pallas import tpu_sc as plsc

EMB_DIM = 128
N_NEG = 5
NC = 2
NS = 16
L = 16
NW = NC * NS
CH = 16
NBUF = 8


def _sc_scores(target, context, neg_flat, syn0, syn1):
    B = target.shape[0]
    BW = B // NW
    nchunk = BW // CH
    mesh = plsc.VectorSubcoreMesh(core_axis_name="c", subcore_axis_name="s")

    @functools.partial(
        pl.kernel,
        out_type=jax.ShapeDtypeStruct((NW, L), jnp.float32),
        mesh=mesh,
        compiler_params=pltpu.CompilerParams(needs_layout_passes=False),
        scratch_types=[
            pltpu.VMEM((BW,), jnp.int32),
            pltpu.VMEM((BW,), jnp.int32),
            pltpu.VMEM((N_NEG, BW), jnp.int32),
            [pltpu.VMEM((CH * N_NEG,), jnp.int32)] * NBUF,
            [pltpu.VMEM((CH, EMB_DIM), jnp.float32)] * NBUF,
            [pltpu.VMEM((CH, EMB_DIM), jnp.float32)] * NBUF,
            [pltpu.VMEM((CH * N_NEG, EMB_DIM), jnp.float32)] * NBUF,
            pltpu.VMEM((L,), jnp.float32),
            [pltpu.SemaphoreType.DMA] * NBUF,
        ],
    )
    def k(t_hbm, c_hbm, n_hbm, syn0_hbm, syn1_hbm, out_hbm,
          t_idx, c_idx, n_idx2, nfs, ts, cs, ns, acc_v, sems):
        wid = lax.axis_index("s") * NC + lax.axis_index("c")
        base = wid * BW
        pltpu.sync_copy(t_hbm.at[pl.ds(base, BW)], t_idx)
        pltpu.sync_copy(c_hbm.at[pl.ds(base, BW)], c_idx)
        pltpu.sync_copy(n_hbm.at[:, pl.ds(base, BW)], n_idx2)

        lane = lax.iota(jnp.int32, L)

        def copies(ch, t_r, c_r, n_r, n_f, sem):
            e0 = ch * CH
            return (
                pltpu.make_async_copy(
                    syn0_hbm.at[t_idx.at[pl.ds(e0, CH)]], t_r, sem),
                pltpu.make_async_copy(
                    syn1_hbm.at[c_idx.at[pl.ds(e0, CH)]], c_r, sem),
                pltpu.make_async_copy(syn1_hbm.at[n_f], n_r, sem),
            )

        def issue(ch, t_r, c_r, n_r, n_f, sem):
            e0 = ch * CH
            for j in range(N_NEG):
                n_f[pl.ds(j * CH, CH)] = n_idx2[j, pl.ds(e0, CH)]
            for cp in copies(ch, t_r, c_r, n_r, n_f, sem):
                cp.start()

        def drain(ch, t_r, c_r, n_r, n_f, sem):
            for cp in copies(ch, t_r, c_r, n_r, n_f, sem):
                cp.wait()

        def log_sigmoid(x):
            z = jnp.exp(-jnp.abs(x))
            s = z / (2.0 + z)
            s2 = s * s
            p = 1.0 / 7.0 + s2 * (1.0 / 9.0)
            p = 1.0 / 5.0 + s2 * p
            p = 1.0 / 3.0 + s2 * p
            log1p_z = 2.0 * s * (1.0 + s2 * p)
            return jnp.minimum(x, 0.0) - log1p_z

        def compute(ch, t_rows, c_rows, n_rows):
            def elem_body(i, svecs):
                tv = [t_rows[i, pl.ds(16 * q, 16)] for q in range(EMB_DIM // L)]

                def dot(rows_ref, r):
                    acc = tv[0] * rows_ref[r, pl.ds(0, 16)]
                    for q in range(1, EMB_DIM // L):
                        acc = acc + tv[q] * rows_ref[r, pl.ds(16 * q, 16)]
                    return jnp.sum(acc)

                m = lane == i
                out = [jnp.where(m, dot(c_rows, i), svecs[0])]
                for j in range(N_NEG):
                    out.append(
                        jnp.where(m, -dot(n_rows, j * CH + i), svecs[1 + j]))
                return out

            svecs = lax.fori_loop(
                0, CH, elem_body, [jnp.zeros((L,), jnp.float32)] * 6)
            part = log_sigmoid(svecs[0])
            for j in range(N_NEG):
                part = part + log_sigmoid(svecs[1 + j])
            return part

        for b in range(NBUF):
            issue(b, ts[b], cs[b], ns[b], nfs[b], sems[b])

        def pipe_body(g, carry):
            for b in range(NBUF):
                ch = NBUF * g + b
                drain(ch, ts[b], cs[b], ns[b], nfs[b], sems[b])
                carry = carry + compute(ch, ts[b], cs[b], ns[b])

                @pl.when(g < nchunk // NBUF - 1)
                def _():
                    issue(ch + NBUF, ts[b], cs[b], ns[b], nfs[b], sems[b])

            return carry

        acc = lax.fori_loop(
            0, nchunk // NBUF, pipe_body, jnp.zeros((L,), jnp.float32))
        acc_v[...] = acc
        pltpu.sync_copy(acc_v, out_hbm.at[wid])

    return k(target, context, neg_flat, syn0, syn1)


def _tc_loss(partials):
    def body(x_ref, o_ref):
        o_ref[...] = jnp.full((1, 1), -jnp.sum(x_ref[...]), jnp.float32)

    return pl.pallas_call(
        body,
        out_shape=jax.ShapeDtypeStruct((1, 1), jnp.float32),
    )(partials)


def kernel(target, context, negatives, syn0, syn1):
    partials = _sc_scores(
        target.astype(jnp.int32),
        context.astype(jnp.int32),
        jnp.swapaxes(negatives, 0, 1).astype(jnp.int32),
        syn0, syn1)
    return jnp.reshape(_tc_loss(partials), ())


# speedup vs baseline: 1.1151x; 1.1151x over previous
import functools

import jax
import jax.numpy as jnp
from jax import lax
from jax.experimental import pallas as pl
from jax.experimental.pallas import tpu as pltpu
from jax.experimental.pallas import tpu_sc as plsc

EMB_DIM = 128
N_NEG = 5
NC = 2
NS = 16
L = 16
NW = NC * NS
CH = 16
NBUF = 4


def _sc_scores(target, context, neg_flat, syn0, syn1):
    B = target.shape[0]
    BW = B // NW
    nchunk = BW // CH
    mesh = plsc.VectorSubcoreMesh(core_axis_name="c", subcore_axis_name="s")

    @functools.partial(
        pl.kernel,
        out_type=jax.ShapeDtypeStruct((NW, L), jnp.float32),
        mesh=mesh,
        compiler_params=pltpu.CompilerParams(needs_layout_passes=False),
        scratch_types=[
            pltpu.VMEM((BW,), jnp.int32),
            pltpu.VMEM((BW,), jnp.int32),
            pltpu.VMEM((N_NEG, BW), jnp.int32),
            [pltpu.VMEM((CH * N_NEG,), jnp.int32)] * NBUF,
            [pltpu.VMEM((CH, EMB_DIM), jnp.float32)] * NBUF,
            [pltpu.VMEM((CH, EMB_DIM), jnp.float32)] * NBUF,
            [pltpu.VMEM((CH * N_NEG, EMB_DIM), jnp.float32)] * NBUF,
            pltpu.VMEM((L,), jnp.float32),
            [pltpu.SemaphoreType.DMA] * NBUF,
        ],
    )
    def k(t_hbm, c_hbm, n_hbm, syn0_hbm, syn1_hbm, out_hbm,
          t_idx, c_idx, n_idx2, nfs, ts, cs, ns, acc_v, sems):
        wid = lax.axis_index("s") * NC + lax.axis_index("c")
        base = wid * BW
        pltpu.sync_copy(t_hbm.at[pl.ds(base, BW)], t_idx)
        pltpu.sync_copy(c_hbm.at[pl.ds(base, BW)], c_idx)
        pltpu.sync_copy(n_hbm.at[:, pl.ds(base, BW)], n_idx2)

        lane = lax.iota(jnp.int32, L)

        def copies(ch, t_r, c_r, n_r, n_f, sem):
            e0 = ch * CH
            return (
                pltpu.make_async_copy(
                    syn0_hbm.at[t_idx.at[pl.ds(e0, CH)]], t_r, sem),
                pltpu.make_async_copy(
                    syn1_hbm.at[c_idx.at[pl.ds(e0, CH)]], c_r, sem),
                pltpu.make_async_copy(syn1_hbm.at[n_f], n_r, sem),
            )

        def issue(ch, t_r, c_r, n_r, n_f, sem):
            e0 = ch * CH
            for j in range(N_NEG):
                n_f[pl.ds(j * CH, CH)] = n_idx2[j, pl.ds(e0, CH)]
            for cp in copies(ch, t_r, c_r, n_r, n_f, sem):
                cp.start()

        def drain(ch, t_r, c_r, n_r, n_f, sem):
            for cp in copies(ch, t_r, c_r, n_r, n_f, sem):
                cp.wait()

        def log_sigmoid(x):
            z = jnp.exp(-jnp.abs(x))
            s = z / (2.0 + z)
            s2 = s * s
            p = 1.0 / 7.0 + s2 * (1.0 / 9.0)
            p = 1.0 / 5.0 + s2 * p
            p = 1.0 / 3.0 + s2 * p
            log1p_z = 2.0 * s * (1.0 + s2 * p)
            return jnp.minimum(x, 0.0) - log1p_z

        def compute(ch, t_rows, c_rows, n_rows):
            def elem_body(i, svecs):
                tv = [t_rows[i, pl.ds(16 * q, 16)] for q in range(EMB_DIM // L)]

                def dot(rows_ref, r):
                    acc = tv[0] * rows_ref[r, pl.ds(0, 16)]
                    for q in range(1, EMB_DIM // L):
                        acc = acc + tv[q] * rows_ref[r, pl.ds(16 * q, 16)]
                    return jnp.sum(acc)

                m = lane == i
                out = [jnp.where(m, dot(c_rows, i), svecs[0])]
                for j in range(N_NEG):
                    out.append(
                        jnp.where(m, -dot(n_rows, j * CH + i), svecs[1 + j]))
                return out

            svecs = lax.fori_loop(
                0, CH, elem_body, [jnp.zeros((L,), jnp.float32)] * 6)
            part = log_sigmoid(svecs[0])
            for j in range(N_NEG):
                part = part + log_sigmoid(svecs[1 + j])
            return part

        for b in range(NBUF):
            issue(b, ts[b], cs[b], ns[b], nfs[b], sems[b])

        def pipe_body(g, carry):
            for b in range(NBUF):
                ch = NBUF * g + b
                drain(ch, ts[b], cs[b], ns[b], nfs[b], sems[b])
                carry = carry + compute(ch, ts[b], cs[b], ns[b])

                @pl.when(g < nchunk // NBUF - 1)
                def _():
                    issue(ch + NBUF, ts[b], cs[b], ns[b], nfs[b], sems[b])

            return carry

        acc = lax.fori_loop(
            0, nchunk // NBUF, pipe_body, jnp.zeros((L,), jnp.float32))
        acc_v[...] = acc
        pltpu.sync_copy(acc_v, out_hbm.at[wid])

    return k(target, context, neg_flat, syn0, syn1)


def _tc_loss(partials):
    def body(x_ref, o_ref):
        o_ref[...] = jnp.full((1, 1), -jnp.sum(x_ref[...]), jnp.float32)

    return pl.pallas_call(
        body,
        out_shape=jax.ShapeDtypeStruct((1, 1), jnp.float32),
    )(partials)


def kernel(target, context, negatives, syn0, syn1):
    partials = _sc_scores(
        target.astype(jnp.int32),
        context.astype(jnp.int32),
        jnp.swapaxes(negatives, 0, 1).astype(jnp.int32),
        syn0, syn1)
    return jnp.reshape(_tc_loss(partials), ())
